# Initial kernel scaffold; baseline (speedup 1.0000x reference)
import jax
import jax.numpy as jnp
from jax.experimental import pallas as pl


def kernel(positions, pe):
    raise NotImplementedError("write your pallas kernel here")


# speedup vs baseline: 4.5265x; 4.5265x over previous
import functools

import jax
import jax.numpy as jnp
from jax import lax
from jax.experimental import pallas as pl
from jax.experimental.pallas import tpu as pltpu
from jax.experimental.pallas import tpu_sc as plsc

D_MODEL = 128
CHUNK = 128


@functools.lru_cache(maxsize=None)
def _make_sc_gather(B):
    info = plsc.get_sparse_core_info()
    nc, ns = info.num_cores, info.num_subcores
    nw = nc * ns
    rows_per_w = B // nw
    steps = rows_per_w // CHUNK
    mesh = plsc.VectorSubcoreMesh(core_axis_name="c", subcore_axis_name="s")

    @functools.partial(
        pl.kernel,
        out_type=jax.ShapeDtypeStruct((B, D_MODEL), jnp.float32),
        mesh=mesh,
        scratch_types=[
            pltpu.VMEM((CHUNK,), jnp.int32),
            pltpu.VMEM((CHUNK, D_MODEL), jnp.float32),
            pltpu.SemaphoreType.DMA,
        ],
    )
    def sc_gather(pos_hbm, pe_hbm, out_hbm, idx_v, rows_v, sem):
        wid = lax.axis_index("s") * nc + lax.axis_index("c")
        base = wid * rows_per_w

        def step(i, carry):
            off = base + i * CHUNK
            pltpu.sync_copy(pos_hbm.at[pl.ds(off, CHUNK)], idx_v)
            pltpu.async_copy(pe_hbm.at[idx_v], rows_v, sem).wait()
            pltpu.sync_copy(rows_v, out_hbm.at[pl.ds(off, CHUNK)])
            return carry

        lax.fori_loop(0, steps, step, 0)

    return sc_gather


def kernel(positions, pe):
    b, t = positions.shape
    pos_flat = positions.reshape(-1).astype(jnp.int32)
    out = _make_sc_gather(pos_flat.shape[0])(pos_flat, pe)
    return out.reshape(b, t, D_MODEL)


# speedup vs baseline: 4.9043x; 1.0835x over previous
import functools

import jax
import jax.numpy as jnp
from jax import lax
from jax.experimental import pallas as pl
from jax.experimental.pallas import tpu as pltpu
from jax.experimental.pallas import tpu_sc as plsc

D_MODEL = 128
CHUNK = 128
BLK_CHUNKS = 2
BLK = BLK_CHUNKS * CHUNK
SB_BLOCKS = 8
SB_CHUNKS = SB_BLOCKS * BLK_CHUNKS


@functools.lru_cache(maxsize=None)
def _make_sc_gather(B):
    info = plsc.get_sparse_core_info()
    nc, ns = info.num_cores, info.num_subcores
    nw = nc * ns
    rows_per_w = B // nw
    n_chunks = B // CHUNK
    w_blocks = rows_per_w // BLK
    w_sbs = w_blocks // SB_BLOCKS
    assert rows_per_w == w_sbs * SB_BLOCKS * BLK
    mesh = plsc.VectorSubcoreMesh(core_axis_name="c", subcore_axis_name="s")

    @functools.partial(
        pl.kernel,
        out_type=jax.ShapeDtypeStruct((B, D_MODEL), jnp.float32),
        mesh=mesh,
        scratch_types=[
            pltpu.VMEM((SB_CHUNKS, CHUNK), jnp.int32),
            pltpu.VMEM((BLK, D_MODEL), jnp.float32),
            pltpu.VMEM((BLK, D_MODEL), jnp.float32),
            pltpu.SemaphoreType.DMA,
            pltpu.SemaphoreType.DMA,
            pltpu.SemaphoreType.DMA,
            pltpu.SemaphoreType.DMA,
        ],
    )
    def sc_gather(pos_hbm, pe_hbm, out_hbm, idx_v, rows_a, rows_b,
                  sg_a, sg_b, ss_a, ss_b):
        wid = lax.axis_index("s") * nc + lax.axis_index("c")
        base_chunk = wid * (rows_per_w // CHUNK)
        base_row = wid * rows_per_w
        bufs = (rows_a, rows_b)
        sgs = (sg_a, sg_b)
        sss = (ss_a, ss_b)

        def wait_gathers(i):
            pltpu.make_async_copy(out_hbm.at[pl.ds(0, BLK)], bufs[i], sgs[i]).wait()

        def wait_scatter(i):
            pltpu.make_async_copy(out_hbm.at[pl.ds(0, BLK)], bufs[i], sss[i]).wait()

        def flush(i, g):
            wait_gathers(i)
            pltpu.async_copy(
                bufs[i], out_hbm.at[pl.ds(base_row + g * BLK, BLK)], sss[i]
            )

        def superblock(sb, carry):
            @pl.when(sb > 0)
            def _():
                flush((SB_BLOCKS - 1) % 2, sb * SB_BLOCKS - 1)

            sb_chunk = base_chunk + sb * SB_CHUNKS
            pltpu.sync_copy(pos_hbm.at[pl.ds(sb_chunk, SB_CHUNKS)], idx_v)

            for blkk in range(SB_BLOCKS):
                cur = blkk % 2
                g = sb * SB_BLOCKS + blkk
                if blkk >= 2:
                    wait_scatter(cur)
                else:
                    @pl.when(sb > 0)
                    def _():
                        wait_scatter(cur)
                for j in range(BLK_CHUNKS):
                    pltpu.async_copy(
                        pe_hbm.at[idx_v.at[blkk * BLK_CHUNKS + j]],
                        bufs[cur].at[pl.ds(j * CHUNK, CHUNK)],
                        sgs[cur],
                    )
                if blkk >= 1:
                    flush(1 - cur, g - 1)
            return carry

        lax.fori_loop(0, w_sbs, superblock, 0)
        last = (SB_BLOCKS - 1) % 2
        flush(last, w_sbs * SB_BLOCKS - 1)
        wait_scatter(1 - last)
        wait_scatter(last)

    return sc_gather


def kernel(positions, pe):
    b, t = positions.shape
    pos2d = positions.reshape(-1, CHUNK).astype(jnp.int32)
    out = _make_sc_gather(b * t)(pos2d, pe)
    return out.reshape(b, t, D_MODEL)


# speedup vs baseline: 18.0984x; 3.6903x over previous
import functools

import jax
import jax.numpy as jnp
from jax import lax
from jax.experimental import pallas as pl
from jax.experimental.pallas import tpu as pltpu
from jax.experimental.pallas import tpu_sc as plsc

D_MODEL = 128
CHUNK = 128
BLK_CHUNKS = 2
BLK = BLK_CHUNKS * CHUNK
SB_BLOCKS = 8
SB_CHUNKS = SB_BLOCKS * BLK_CHUNKS


@functools.lru_cache(maxsize=None)
def _make_sc_gather(B):
    info = plsc.get_sparse_core_info()
    nc, ns = info.num_cores, info.num_subcores
    nw = nc * ns
    rows_per_w = B // nw
    n_chunks = B // CHUNK
    w_blocks = rows_per_w // BLK
    w_sbs = w_blocks // SB_BLOCKS
    assert rows_per_w == w_sbs * SB_BLOCKS * BLK
    mesh = plsc.VectorSubcoreMesh(core_axis_name="c", subcore_axis_name="s")

    @functools.partial(
        pl.kernel,
        out_type=jax.ShapeDtypeStruct((B, D_MODEL), jnp.float32),
        mesh=mesh,
        scratch_types=[
            pltpu.VMEM((SB_CHUNKS, CHUNK), jnp.int32),
            pltpu.VMEM_SHARED((367, D_MODEL), jnp.float32),
            pltpu.VMEM((BLK, D_MODEL), jnp.float32),
            pltpu.VMEM((BLK, D_MODEL), jnp.float32),
            pltpu.SemaphoreType.DMA,
            pltpu.SemaphoreType.DMA,
            pltpu.SemaphoreType.DMA,
            pltpu.SemaphoreType.DMA,
        ],
    )
    def sc_gather(pos_hbm, pe_hbm, out_hbm, idx_v, pe_v, rows_a, rows_b,
                  sg_a, sg_b, ss_a, ss_b):
        @pl.when(lax.axis_index("s") == 0)
        def _():
            pltpu.sync_copy(pe_hbm, pe_v)
        plsc.subcore_barrier()
        wid = lax.axis_index("s") * nc + lax.axis_index("c")
        base_chunk = wid * (rows_per_w // CHUNK)
        base_row = wid * rows_per_w
        bufs = (rows_a, rows_b)
        sgs = (sg_a, sg_b)
        sss = (ss_a, ss_b)

        def wait_gathers(i):
            pltpu.make_async_copy(out_hbm.at[pl.ds(0, BLK)], bufs[i], sgs[i]).wait()

        def wait_scatter(i):
            pltpu.make_async_copy(out_hbm.at[pl.ds(0, BLK)], bufs[i], sss[i]).wait()

        def flush(i, g):
            wait_gathers(i)
            pltpu.async_copy(
                bufs[i], out_hbm.at[pl.ds(base_row + g * BLK, BLK)], sss[i]
            )

        def superblock(sb, carry):
            @pl.when(sb > 0)
            def _():
                flush((SB_BLOCKS - 1) % 2, sb * SB_BLOCKS - 1)

            sb_chunk = base_chunk + sb * SB_CHUNKS
            pltpu.sync_copy(pos_hbm.at[pl.ds(sb_chunk, SB_CHUNKS)], idx_v)

            for blkk in range(SB_BLOCKS):
                cur = blkk % 2
                g = sb * SB_BLOCKS + blkk
                if blkk >= 2:
                    wait_scatter(cur)
                else:
                    @pl.when(sb > 0)
                    def _():
                        wait_scatter(cur)
                for j in range(BLK_CHUNKS):
                    pltpu.async_copy(
                        pe_v.at[idx_v.at[blkk * BLK_CHUNKS + j]],
                        bufs[cur].at[pl.ds(j * CHUNK, CHUNK)],
                        sgs[cur],
                    )
                if blkk >= 1:
                    flush(1 - cur, g - 1)
            return carry

        lax.fori_loop(0, w_sbs, superblock, 0)
        last = (SB_BLOCKS - 1) % 2
        flush(last, w_sbs * SB_BLOCKS - 1)
        wait_scatter(1 - last)
        wait_scatter(last)

    return sc_gather


def kernel(positions, pe):
    b, t = positions.shape
    pos2d = positions.reshape(-1, CHUNK).astype(jnp.int32)
    out = _make_sc_gather(b * t)(pos2d, pe)
    return out.reshape(b, t, D_MODEL)


# speedup vs baseline: 19.4967x; 1.0773x over previous
import functools

import jax
import jax.numpy as jnp
from jax import lax
from jax.experimental import pallas as pl
from jax.experimental.pallas import tpu as pltpu
from jax.experimental.pallas import tpu_sc as plsc

D_MODEL = 128
CHUNK = 128
BLK_CHUNKS = 2
BLK = BLK_CHUNKS * CHUNK
SB_BLOCKS = 4
SB_CHUNKS = SB_BLOCKS * BLK_CHUNKS


@functools.lru_cache(maxsize=None)
def _make_sc_gather(B):
    info = plsc.get_sparse_core_info()
    nc, ns = info.num_cores, info.num_subcores
    nw = nc * ns
    rows_per_w = B // nw
    n_chunks = B // CHUNK
    w_sbs = rows_per_w // (SB_BLOCKS * BLK)
    assert rows_per_w == w_sbs * SB_BLOCKS * BLK and w_sbs % 2 == 0
    mesh = plsc.VectorSubcoreMesh(core_axis_name="c", subcore_axis_name="s")

    @functools.partial(
        pl.kernel,
        out_type=jax.ShapeDtypeStruct((B, D_MODEL), jnp.float32),
        mesh=mesh,
        scratch_types=[
            pltpu.VMEM((SB_CHUNKS, CHUNK), jnp.int32),
            pltpu.VMEM((SB_CHUNKS, CHUNK), jnp.int32),
            pltpu.VMEM_SHARED((367, D_MODEL), jnp.float32),
            pltpu.VMEM((BLK, D_MODEL), jnp.float32),
            pltpu.VMEM((BLK, D_MODEL), jnp.float32),
            pltpu.SemaphoreType.DMA,
            pltpu.SemaphoreType.DMA,
            pltpu.SemaphoreType.DMA,
            pltpu.SemaphoreType.DMA,
            pltpu.SemaphoreType.DMA,
            pltpu.SemaphoreType.DMA,
        ],
    )
    def sc_gather(pos_hbm, pe_hbm, out_hbm, idx_a, idx_b, pe_v, rows_a, rows_b,
                  sg_a, sg_b, ss_a, ss_b, si_a, si_b):
        @pl.when(lax.axis_index("s") == 0)
        def _():
            pltpu.sync_copy(pe_hbm, pe_v)
        plsc.subcore_barrier()

        wid = lax.axis_index("s") * nc + lax.axis_index("c")
        base_chunk = wid * (rows_per_w // CHUNK)
        base_row = wid * rows_per_w
        bufs = (rows_a, rows_b)
        sgs = (sg_a, sg_b)
        sss = (ss_a, ss_b)
        idxs = (idx_a, idx_b)
        sis = (si_a, si_b)

        def prefetch_idx(sb, p):
            off = jnp.minimum(base_chunk + sb * SB_CHUNKS, n_chunks - SB_CHUNKS)
            pltpu.async_copy(pos_hbm.at[pl.ds(off, SB_CHUNKS)], idxs[p], sis[p])

        def wait_idx(p):
            pltpu.make_async_copy(
                pos_hbm.at[pl.ds(0, SB_CHUNKS)], idxs[p], sis[p]
            ).wait()

        def wait_gathers(i):
            pltpu.make_async_copy(out_hbm.at[pl.ds(0, BLK)], bufs[i], sgs[i]).wait()

        def wait_scatter(i):
            pltpu.make_async_copy(out_hbm.at[pl.ds(0, BLK)], bufs[i], sss[i]).wait()

        def flush(i, g):
            wait_gathers(i)
            pltpu.async_copy(
                bufs[i], out_hbm.at[pl.ds(base_row + g * BLK, BLK)], sss[i]
            )

        def do_sb(sb, p):
            wait_idx(p)
            for blkk in range(SB_BLOCKS):
                cur = blkk % 2
                g = sb * SB_BLOCKS + blkk
                if blkk >= 2:
                    wait_scatter(cur)
                else:
                    @pl.when(sb > 0)
                    def _():
                        wait_scatter(cur)
                for j in range(BLK_CHUNKS):
                    pltpu.async_copy(
                        pe_v.at[idxs[p].at[blkk * BLK_CHUNKS + j]],
                        bufs[cur].at[pl.ds(j * CHUNK, CHUNK)],
                        sgs[cur],
                    )
                if blkk >= 1:
                    flush(1 - cur, g - 1)
                else:
                    @pl.when(sb > 0)
                    def _():
                        flush(1 - cur, g - 1)
                if blkk == 0:
                    prefetch_idx(sb + 1, 1 - p)

        prefetch_idx(0, 0)

        def pair(pp, carry):
            do_sb(2 * pp, 0)
            do_sb(2 * pp + 1, 1)
            return carry

        lax.fori_loop(0, w_sbs // 2, pair, 0)
        last = (SB_BLOCKS - 1) % 2
        flush(last, w_sbs * SB_BLOCKS - 1)
        wait_idx(0)
        wait_scatter(1 - last)
        wait_scatter(last)

    return sc_gather


def kernel(positions, pe):
    b, t = positions.shape
    pos2d = positions.reshape(-1, CHUNK).astype(jnp.int32)
    out = _make_sc_gather(b * t)(pos2d, pe)
    return out.reshape(b, t, D_MODEL)
